# baseline (speedup 1.0000x reference)
import jax
import jax.numpy as jnp
from jax.experimental import pallas as pl
from jax.experimental.pallas import tpu as pltpu

L2_NORM_CLIP = 1.0

BLOCK_ROWS = 4096


def _clip_add_body(x_ref, noise_ref, out_ref):
    x = x_ref[...]
    ssq = jnp.sum(x * x, axis=1, keepdims=True)
    scale = jax.lax.rsqrt(jnp.maximum(ssq, L2_NORM_CLIP * L2_NORM_CLIP)) * L2_NORM_CLIP
    out_ref[...] = x * scale + noise_ref[...]


def kernel(x, noise):
    n, d = x.shape
    grid = (n // BLOCK_ROWS,)
    return pl.pallas_call(
        _clip_add_body,
        grid=grid,
        in_specs=[
            pl.BlockSpec((BLOCK_ROWS, d), lambda i: (i, 0)),
            pl.BlockSpec((BLOCK_ROWS, d), lambda i: (i, 0)),
        ],
        out_specs=pl.BlockSpec((BLOCK_ROWS, d), lambda i: (i, 0)),
        out_shape=jax.ShapeDtypeStruct((n, d), x.dtype),
        compiler_params=pltpu.CompilerParams(
            dimension_semantics=("parallel",),
        ),
    )(x, noise)


# speedup vs baseline: 1.0616x; 1.0616x over previous
import jax
import jax.numpy as jnp
from jax.experimental import pallas as pl
from jax.experimental.pallas import tpu as pltpu

L2_NORM_CLIP = 1.0

BLOCK_ROWS = 16384


def _clip_add_body(x_ref, noise_ref, out_ref):
    x = x_ref[...]
    ssq = jnp.sum(x * x, axis=1, keepdims=True)
    scale = jax.lax.rsqrt(jnp.maximum(ssq, L2_NORM_CLIP * L2_NORM_CLIP)) * L2_NORM_CLIP
    out_ref[...] = x * scale + noise_ref[...]


def kernel(x, noise):
    n, d = x.shape
    grid = (n // BLOCK_ROWS,)
    return pl.pallas_call(
        _clip_add_body,
        grid=grid,
        in_specs=[
            pl.BlockSpec((BLOCK_ROWS, d), lambda i: (i, 0)),
            pl.BlockSpec((BLOCK_ROWS, d), lambda i: (i, 0)),
        ],
        out_specs=pl.BlockSpec((BLOCK_ROWS, d), lambda i: (i, 0)),
        out_shape=jax.ShapeDtypeStruct((n, d), x.dtype),
        compiler_params=pltpu.CompilerParams(
            dimension_semantics=("parallel",),
        ),
    )(x, noise)
